# Initial kernel scaffold; baseline (speedup 1.0000x reference)
import jax
import jax.numpy as jnp
from jax.experimental import pallas as pl


def kernel(node_src_feats, node_tgt_feats, edge_ids):
    raise NotImplementedError("write your pallas kernel here")


# speedup vs baseline: 2.4274x; 2.4274x over previous
import jax
import jax.numpy as jnp
from jax import lax
from jax.experimental import pallas as pl
from jax.experimental.pallas import tpu as pltpu
from jax.experimental.pallas import tpu_sc as plsc

E = 320000
D = 128
C = 128
NCHUNKS = E // C
NW = 32


def _sc_body(src_hbm, tgt_hbm, sidx_hbm, tidx_hbm, out_hbm,
             sidx_v, tidx_v, a_v, b_v, sem_s, sem_t):
    cid = lax.axis_index("c")
    sid = lax.axis_index("s")
    wid = sid * 2 + cid

    base_chunks = NCHUNKS // NW
    extra = NCHUNKS - base_chunks * NW
    nch = base_chunks + jnp.where(wid < extra, 1, 0)
    start = wid * base_chunks + jnp.minimum(wid, extra)

    def chunk_body(i, carry):
        base = (start + i) * C
        pltpu.sync_copy(sidx_hbm.at[pl.ds(base, C)], sidx_v)
        pltpu.sync_copy(tidx_hbm.at[pl.ds(base, C)], tidx_v)
        cp_s = pltpu.async_copy(src_hbm.at[sidx_v], a_v, sem_s)
        cp_t = pltpu.async_copy(tgt_hbm.at[tidx_v], b_v, sem_t)
        cp_s.wait()
        cp_t.wait()

        def row_body(r, c2):
            for j in range(D // 16):
                sl = pl.ds(j * 16, 16)
                a_v[r, sl] = a_v[r, sl] * b_v[r, sl]
            return c2

        lax.fori_loop(0, C, row_body, 0, unroll=2)
        pltpu.sync_copy(a_v, out_hbm.at[pl.ds(base, C)])
        return carry

    lax.fori_loop(0, nch, chunk_body, 0)


def kernel(node_src_feats, node_tgt_feats, edge_ids):
    eids = edge_ids.astype(jnp.int32)
    sidx = eids[0]
    tidx = eids[1]

    mesh = plsc.VectorSubcoreMesh(core_axis_name="c", subcore_axis_name="s")
    f = pl.kernel(
        _sc_body,
        mesh=mesh,
        out_type=jax.ShapeDtypeStruct((E, D), jnp.float32),
        scratch_types=[
            pltpu.VMEM((C,), jnp.int32),
            pltpu.VMEM((C,), jnp.int32),
            pltpu.VMEM((C, D), jnp.float32),
            pltpu.VMEM((C, D), jnp.float32),
            pltpu.SemaphoreType.DMA,
            pltpu.SemaphoreType.DMA,
        ],
    )
    return f(node_src_feats, node_tgt_feats, sidx, tidx)


# speedup vs baseline: 4.2930x; 1.7686x over previous
import jax
import jax.numpy as jnp
from jax import lax
from jax.experimental import pallas as pl
from jax.experimental.pallas import tpu as pltpu
from jax.experimental.pallas import tpu_sc as plsc

E = 320000
D = 128
NW = 32
C = 40
CH = E // (NW * C)
NB = 5
ROUNDS = CH // NB


def _sc_body(src_hbm, tgt_hbm, sidx_hbm, tidx_hbm, out_hbm,
             sidx_v, tidx_v, a_v, b_v, o_v,
             sem_g0, sem_g1, sem_g2, sem_g3, sem_g4,
             sem_o0, sem_o1, sem_o2, sem_o3, sem_o4):
    cid = lax.axis_index("c")
    sid = lax.axis_index("s")
    wid = sid * 2 + cid
    row0 = wid * (CH * C)

    sem_g = [sem_g0, sem_g1, sem_g2, sem_g3, sem_g4]
    sem_o = [sem_o0, sem_o1, sem_o2, sem_o3, sem_o4]

    pltpu.sync_copy(sidx_hbm.at[wid], sidx_v)
    pltpu.sync_copy(tidx_hbm.at[wid], tidx_v)

    def fire(chunk, b):
        sl = pl.ds(chunk * C, C)
        pltpu.async_copy(src_hbm.at[sidx_v.at[sl]], a_v.at[b], sem_g[b])
        pltpu.async_copy(tgt_hbm.at[tidx_v.at[sl]], b_v.at[b], sem_g[b])

    def wait_gathers(chunk, b):
        sl = pl.ds(chunk * C, C)
        pltpu.make_async_copy(src_hbm.at[sidx_v.at[sl]], a_v.at[b],
                              sem_g[b]).wait()
        pltpu.make_async_copy(tgt_hbm.at[tidx_v.at[sl]], b_v.at[b],
                              sem_g[b]).wait()

    def wait_out(chunk, b):
        pltpu.make_async_copy(o_v.at[b], out_hbm.at[pl.ds(row0 + chunk * C, C)],
                              sem_o[b]).wait()

    for b in range(NB):
        fire(b, b)

    def round_body(r, carry):
        base = r * NB
        for b in range(NB):
            chunk = base + b

            @pl.when(r >= 1)
            def _():
                wait_out(chunk - NB, b)

            wait_gathers(chunk, b)

            def row_body(row, c2):
                for j in range(D // 16):
                    sl = pl.ds(j * 16, 16)
                    o_v[b, row, sl] = a_v[b, row, sl] * b_v[b, row, sl]
                return c2

            lax.fori_loop(0, C, row_body, 0, unroll=2)

            @pl.when(r < ROUNDS - 1)
            def _():
                fire(chunk + NB, b)

            pltpu.async_copy(o_v.at[b],
                             out_hbm.at[pl.ds(row0 + chunk * C, C)], sem_o[b])
        return carry

    lax.fori_loop(0, ROUNDS, round_body, 0)

    for b in range(NB):
        wait_out((ROUNDS - 1) * NB + b, b)


def kernel(node_src_feats, node_tgt_feats, edge_ids):
    eids = edge_ids.astype(jnp.int32)
    sidx = eids[0].reshape(NW, CH * C)
    tidx = eids[1].reshape(NW, CH * C)

    mesh = plsc.VectorSubcoreMesh(core_axis_name="c", subcore_axis_name="s")
    f = pl.kernel(
        _sc_body,
        mesh=mesh,
        out_type=jax.ShapeDtypeStruct((E, D), jnp.float32),
        scratch_types=[
            pltpu.VMEM((CH * C,), jnp.int32),
            pltpu.VMEM((CH * C,), jnp.int32),
            pltpu.VMEM((NB, C, D), jnp.float32),
            pltpu.VMEM((NB, C, D), jnp.float32),
            pltpu.VMEM((NB, C, D), jnp.float32),
        ] + [pltpu.SemaphoreType.DMA] * (2 * NB),
    )
    return f(node_src_feats, node_tgt_feats, sidx, tidx)


# speedup vs baseline: 5.7809x; 1.3466x over previous
import jax
import jax.numpy as jnp
from jax import lax
from jax.experimental import pallas as pl
from jax.experimental.pallas import tpu as pltpu
from jax.experimental.pallas import tpu_sc as plsc

E = 320000
D = 128
NW = 32
C = 40
CH = E // (NW * C)
NB = 5
ROUNDS = CH // NB


def _sc_body(tab_hbm, idx_hbm, out_hbm,
             idx_v, a_v, o_v,
             sem_g0, sem_g1, sem_g2, sem_g3, sem_g4,
             sem_o0, sem_o1, sem_o2, sem_o3, sem_o4):
    cid = lax.axis_index("c")
    sid = lax.axis_index("s")
    wid = sid * 2 + cid
    row0 = wid * (CH * C)

    sem_g = [sem_g0, sem_g1, sem_g2, sem_g3, sem_g4]
    sem_o = [sem_o0, sem_o1, sem_o2, sem_o3, sem_o4]

    pltpu.sync_copy(idx_hbm.at[wid], idx_v)

    def fire(chunk, b):
        sl = pl.ds(chunk * 2 * C, 2 * C)
        pltpu.async_copy(tab_hbm.at[idx_v.at[sl]], a_v.at[b], sem_g[b])

    def wait_gather(chunk, b):
        sl = pl.ds(chunk * 2 * C, 2 * C)
        pltpu.make_async_copy(tab_hbm.at[idx_v.at[sl]], a_v.at[b],
                              sem_g[b]).wait()

    def wait_out(chunk, b):
        pltpu.make_async_copy(o_v.at[b], out_hbm.at[pl.ds(row0 + chunk * C, C)],
                              sem_o[b]).wait()

    for b in range(NB):
        fire(b, b)

    def round_body(r, carry):
        base = r * NB
        for b in range(NB):
            chunk = base + b

            @pl.when(r >= 1)
            def _():
                wait_out(chunk - NB, b)

            wait_gather(chunk, b)

            def row_body(row, c2):
                for j in range(D // 16):
                    sl = pl.ds(j * 16, 16)
                    o_v[b, row, sl] = a_v[b, row, sl] * a_v[b, C + row, sl]
                return c2

            lax.fori_loop(0, C, row_body, 0, unroll=2)

            @pl.when(r < ROUNDS - 1)
            def _():
                fire(chunk + NB, b)

            pltpu.async_copy(o_v.at[b],
                             out_hbm.at[pl.ds(row0 + chunk * C, C)], sem_o[b])
        return carry

    lax.fori_loop(0, ROUNDS, round_body, 0)

    for b in range(NB):
        wait_out((ROUNDS - 1) * NB + b, b)


def kernel(node_src_feats, node_tgt_feats, edge_ids):
    table = jnp.concatenate([node_src_feats, node_tgt_feats], axis=0)
    eids = edge_ids.astype(jnp.int32)
    sidx = eids[0].reshape(NW, CH, C)
    tidx = eids[1].reshape(NW, CH, C) + node_src_feats.shape[0]
    idx = jnp.concatenate([sidx, tidx], axis=2).reshape(NW, CH * 2 * C)

    mesh = plsc.VectorSubcoreMesh(core_axis_name="c", subcore_axis_name="s")
    f = pl.kernel(
        _sc_body,
        mesh=mesh,
        out_type=jax.ShapeDtypeStruct((E, D), jnp.float32),
        scratch_types=[
            pltpu.VMEM((CH * 2 * C,), jnp.int32),
            pltpu.VMEM((NB, 2 * C, D), jnp.float32),
            pltpu.VMEM((NB, C, D), jnp.float32),
        ] + [pltpu.SemaphoreType.DMA] * (2 * NB),
    )
    return f(table, idx)


# speedup vs baseline: 7.9762x; 1.3798x over previous
import jax
import jax.numpy as jnp
from jax import lax
from jax.experimental import pallas as pl
from jax.experimental.pallas import tpu as pltpu
from jax.experimental.pallas import tpu_sc as plsc

E = 320000
D = 128
NW = 32
C = 80
CH = E // (NW * C)
NB = 5
LEAD = 3
ROUNDS = CH // NB


def _sc_body(src_hbm, tgt_hbm, sidx_hbm, tidx_hbm, out_hbm,
             sidx_v, tidx_v, a_v,
             sem_g0, sem_g1, sem_g2, sem_g3, sem_g4,
             sem_o0, sem_o1, sem_o2, sem_o3, sem_o4):
    cid = lax.axis_index("c")
    sid = lax.axis_index("s")
    wid = sid * 2 + cid
    row0 = wid * (CH * C)

    sem_g = [sem_g0, sem_g1, sem_g2, sem_g3, sem_g4]
    sem_o = [sem_o0, sem_o1, sem_o2, sem_o3, sem_o4]

    pltpu.sync_copy(sidx_hbm.at[wid], sidx_v)
    pltpu.sync_copy(tidx_hbm.at[wid], tidx_v)

    def fire(chunk, b):
        sl = pl.ds(chunk * C, C)
        pltpu.async_copy(src_hbm.at[sidx_v.at[sl]],
                         a_v.at[b, pl.ds(0, C)], sem_g[b])
        pltpu.async_copy(tgt_hbm.at[tidx_v.at[sl]],
                         a_v.at[b, pl.ds(C, C)], sem_g[b])

    def wait_gathers(chunk, b):
        sl = pl.ds(chunk * C, C)
        pltpu.make_async_copy(src_hbm.at[sidx_v.at[sl]],
                              a_v.at[b, pl.ds(0, C)], sem_g[b]).wait()
        pltpu.make_async_copy(tgt_hbm.at[tidx_v.at[sl]],
                              a_v.at[b, pl.ds(C, C)], sem_g[b]).wait()

    def wait_out(chunk, b):
        pltpu.make_async_copy(a_v.at[b, pl.ds(0, C)],
                              out_hbm.at[pl.ds(row0 + chunk * C, C)],
                              sem_o[b]).wait()

    for b in range(LEAD):
        fire(b, b)

    def round_body(r, carry):
        base = r * NB
        for b in range(NB):
            chunk = base + b

            wait_gathers(chunk, b)

            def row_body(row, c2):
                for j in range(D // 16):
                    sl = pl.ds(j * 16, 16)
                    a_v[b, row, sl] = a_v[b, row, sl] * a_v[b, C + row, sl]
                return c2

            lax.fori_loop(0, C, row_body, 0, unroll=2)

            pltpu.async_copy(a_v.at[b, pl.ds(0, C)],
                             out_hbm.at[pl.ds(row0 + chunk * C, C)], sem_o[b])

            nb_ = (b + LEAD) % NB

            @pl.when(chunk < CH - LEAD)
            def _():
                @pl.when(chunk >= NB - LEAD)
                def _():
                    wait_out(chunk + LEAD - NB, nb_)
                fire(chunk + LEAD, nb_)
        return carry

    lax.fori_loop(0, ROUNDS, round_body, 0)

    for k in range(NB):
        chunk = CH - NB + k
        wait_out(chunk, chunk % NB)


def kernel(node_src_feats, node_tgt_feats, edge_ids):
    eids = edge_ids.astype(jnp.int32)
    sidx = eids[0].reshape(NW, CH * C)
    tidx = eids[1].reshape(NW, CH * C)

    mesh = plsc.VectorSubcoreMesh(core_axis_name="c", subcore_axis_name="s")
    f = pl.kernel(
        _sc_body,
        mesh=mesh,
        out_type=jax.ShapeDtypeStruct((E, D), jnp.float32),
        scratch_types=[
            pltpu.VMEM((CH * C,), jnp.int32),
            pltpu.VMEM((CH * C,), jnp.int32),
            pltpu.VMEM((NB, 2 * C, D), jnp.float32),
        ] + [pltpu.SemaphoreType.DMA] * (2 * NB),
    )
    return f(node_src_feats, node_tgt_feats, sidx, tidx)


# speedup vs baseline: 8.4377x; 1.0579x over previous
import jax
import jax.numpy as jnp
from jax import lax
from jax.experimental import pallas as pl
from jax.experimental.pallas import tpu as pltpu
from jax.experimental.pallas import tpu_sc as plsc

E = 320000
D = 128
NW = 32
C = 80
CH = E // (NW * C)
NB = 5
LEAD = 3
ROUNDS = CH // NB


def _sc_body(src_hbm, tgt_hbm, eidx_hbm, out_hbm,
             sidx_v, tidx_v, a_v,
             sem_g0, sem_g1, sem_g2, sem_g3, sem_g4,
             sem_o0, sem_o1, sem_o2, sem_o3, sem_o4):
    cid = lax.axis_index("c")
    sid = lax.axis_index("s")
    wid = sid * 2 + cid
    row0 = wid * (CH * C)

    sem_g = [sem_g0, sem_g1, sem_g2, sem_g3, sem_g4]
    sem_o = [sem_o0, sem_o1, sem_o2, sem_o3, sem_o4]

    pltpu.sync_copy(eidx_hbm.at[pl.ds(wid * (CH * C), CH * C)], sidx_v)
    pltpu.sync_copy(eidx_hbm.at[pl.ds(E + wid * (CH * C), CH * C)], tidx_v)

    def fire(chunk, b):
        sl = pl.ds(chunk * C, C)
        pltpu.async_copy(src_hbm.at[sidx_v.at[sl]],
                         a_v.at[b, pl.ds(0, C)], sem_g[b])
        pltpu.async_copy(tgt_hbm.at[tidx_v.at[sl]],
                         a_v.at[b, pl.ds(C, C)], sem_g[b])

    def wait_gathers(chunk, b):
        sl = pl.ds(chunk * C, C)
        pltpu.make_async_copy(src_hbm.at[sidx_v.at[sl]],
                              a_v.at[b, pl.ds(0, C)], sem_g[b]).wait()
        pltpu.make_async_copy(tgt_hbm.at[tidx_v.at[sl]],
                              a_v.at[b, pl.ds(C, C)], sem_g[b]).wait()

    def wait_out(chunk, b):
        pltpu.make_async_copy(a_v.at[b, pl.ds(0, C)],
                              out_hbm.at[pl.ds(row0 + chunk * C, C)],
                              sem_o[b]).wait()

    for b in range(LEAD):
        fire(b, b)

    def round_body(r, carry):
        base = r * NB
        for b in range(NB):
            chunk = base + b

            wait_gathers(chunk, b)

            def row_body(row, c2):
                for j in range(D // 16):
                    sl = pl.ds(j * 16, 16)
                    a_v[b, row, sl] = a_v[b, row, sl] * a_v[b, C + row, sl]
                return c2

            lax.fori_loop(0, C, row_body, 0, unroll=2)

            pltpu.async_copy(a_v.at[b, pl.ds(0, C)],
                             out_hbm.at[pl.ds(row0 + chunk * C, C)], sem_o[b])

            nb_ = (b + LEAD) % NB

            @pl.when(chunk < CH - LEAD)
            def _():
                @pl.when(chunk >= NB - LEAD)
                def _():
                    wait_out(chunk + LEAD - NB, nb_)
                fire(chunk + LEAD, nb_)
        return carry

    lax.fori_loop(0, ROUNDS, round_body, 0)

    for k in range(NB):
        chunk = CH - NB + k
        wait_out(chunk, chunk % NB)


def kernel(node_src_feats, node_tgt_feats, edge_ids):
    eids = edge_ids.astype(jnp.int32).reshape(2 * E)

    mesh = plsc.VectorSubcoreMesh(core_axis_name="c", subcore_axis_name="s")
    f = pl.kernel(
        _sc_body,
        mesh=mesh,
        out_type=jax.ShapeDtypeStruct((E, D), jnp.float32),
        scratch_types=[
            pltpu.VMEM((CH * C,), jnp.int32),
            pltpu.VMEM((CH * C,), jnp.int32),
            pltpu.VMEM((NB, 2 * C, D), jnp.float32),
        ] + [pltpu.SemaphoreType.DMA] * (2 * NB),
    )
    return f(node_src_feats, node_tgt_feats, eids)


# speedup vs baseline: 8.4499x; 1.0014x over previous
import jax
import jax.numpy as jnp
from jax import lax
from jax.experimental import pallas as pl
from jax.experimental.pallas import tpu as pltpu
from jax.experimental.pallas import tpu_sc as plsc

E = 320000
D = 128
NW = 32
C = 80
CH = E // (NW * C)
NB = 5
LEAD = 3
ROUNDS = CH // NB


def _sc_body(src_hbm, tgt_hbm, eidx_hbm, out_hbm,
             sidx_v, tidx_v, a_v,
             sem_g0, sem_g1, sem_g2, sem_g3, sem_g4,
             sem_o0, sem_o1, sem_o2, sem_o3, sem_o4):
    cid = lax.axis_index("c")
    sid = lax.axis_index("s")
    wid = sid * 2 + cid
    row0 = wid * (CH * C)

    sem_g = [sem_g0, sem_g1, sem_g2, sem_g3, sem_g4]
    sem_o = [sem_o0, sem_o1, sem_o2, sem_o3, sem_o4]

    pltpu.sync_copy(eidx_hbm.at[pl.ds(wid * (CH * C), CH * C)], sidx_v)
    pltpu.sync_copy(eidx_hbm.at[pl.ds(E + wid * (CH * C), CH * C)], tidx_v)

    def fire(chunk, b):
        sl = pl.ds(chunk * C, C)
        pltpu.async_copy(src_hbm.at[sidx_v.at[sl]],
                         a_v.at[b, pl.ds(0, C)], sem_g[b])
        pltpu.async_copy(tgt_hbm.at[tidx_v.at[sl]],
                         a_v.at[b, pl.ds(C, C)], sem_g[b])

    def wait_gathers(chunk, b):
        sl = pl.ds(chunk * C, C)
        pltpu.make_async_copy(src_hbm.at[sidx_v.at[sl]],
                              a_v.at[b, pl.ds(0, C)], sem_g[b]).wait()
        pltpu.make_async_copy(tgt_hbm.at[tidx_v.at[sl]],
                              a_v.at[b, pl.ds(C, C)], sem_g[b]).wait()

    def wait_out(chunk, b):
        pltpu.make_async_copy(a_v.at[b, pl.ds(0, C)],
                              out_hbm.at[pl.ds(row0 + chunk * C, C)],
                              sem_o[b]).wait()

    for b in range(LEAD):
        fire(b, b)

    def round_body(r, carry):
        base = r * NB
        for b in range(NB):
            chunk = base + b

            wait_gathers(chunk, b)

            def row_body(row, c2):
                for j in range(D // 16):
                    sl = pl.ds(j * 16, 16)
                    a_v[b, row, sl] = a_v[b, row, sl] * a_v[b, C + row, sl]
                return c2

            lax.fori_loop(0, C, row_body, 0, unroll=4)

            pltpu.async_copy(a_v.at[b, pl.ds(0, C)],
                             out_hbm.at[pl.ds(row0 + chunk * C, C)], sem_o[b])

            nb_ = (b + LEAD) % NB

            @pl.when(chunk < CH - LEAD)
            def _():
                @pl.when(chunk >= NB - LEAD)
                def _():
                    wait_out(chunk + LEAD - NB, nb_)
                fire(chunk + LEAD, nb_)
        return carry

    lax.fori_loop(0, ROUNDS, round_body, 0)

    for k in range(NB):
        chunk = CH - NB + k
        wait_out(chunk, chunk % NB)


def kernel(node_src_feats, node_tgt_feats, edge_ids):
    eids = edge_ids.astype(jnp.int32).reshape(2 * E)

    mesh = plsc.VectorSubcoreMesh(core_axis_name="c", subcore_axis_name="s")
    f = pl.kernel(
        _sc_body,
        mesh=mesh,
        out_type=jax.ShapeDtypeStruct((E, D), jnp.float32),
        scratch_types=[
            pltpu.VMEM((CH * C,), jnp.int32),
            pltpu.VMEM((CH * C,), jnp.int32),
            pltpu.VMEM((NB, 2 * C, D), jnp.float32),
        ] + [pltpu.SemaphoreType.DMA] * (2 * NB),
    )
    return f(node_src_feats, node_tgt_feats, eids)
